# Initial kernel scaffold; baseline (speedup 1.0000x reference)
import jax
import jax.numpy as jnp
from jax.experimental import pallas as pl


def kernel(xs, ds, is_inference):
    raise NotImplementedError("write your pallas kernel here")


# speedup vs baseline: 20.7453x; 20.7453x over previous
import jax
import jax.numpy as jnp
from jax import lax
from jax.experimental import pallas as pl
from jax.experimental.pallas import tpu as pltpu
from jax.experimental.pallas import tpu_sc as plsc

B, T, C, K = 16, 512, 256, 4
NROWS = B * K
WIN = 16
NIT = 256
DT = 1.0 / NIT
L = 16
NC, NS = 2, 16
NPAIR = (B * T) // (NC * NS)
NCHUNK = 8
PCHUNK = NPAIR // NCHUNK


def _sc_body(v_hbm, xs_hbm, ys_hbm, func_hbm,
             vp, vs, phi, f4, func_v, wbuf, idx_all, grows, orow, shared,
             sem):
    c = lax.axis_index("c")
    s = lax.axis_index("s")
    w = c * NS + s
    iot = lax.iota(jnp.int32, L)

    for rowi in range(2):
        r = 2 * w + rowi
        lr = 2 * s + rowi
        pltpu.sync_copy(v_hbm.at[r], vp)

        def smooth_body(j, carry):
            base = j * L + iot - (WIN // 2)
            acc = plsc.load_gather(vp, [jnp.maximum(base, 0)])
            for d in range(1, WIN):
                acc = acc + plsc.load_gather(
                    vp, [jnp.clip(base + d, 0, T - 1)])
            vs[pl.ds(j * L, L)] = acc * (1.0 / (WIN * K))
            return carry
        lax.fori_loop(0, T // L, smooth_body, 0)

        def init_body(j, carry):
            phi[pl.ds(j * L, L)] = (j * L + iot).astype(jnp.float32)
            return carry
        lax.fori_loop(0, T // L, init_body, 0)

        def euler_body(n, carry):
            def grp(j8, c2):
                for u in range(4):
                    off = j8 * (4 * L) + u * L
                    p = phi[pl.ds(off, L)]
                    i0 = jnp.clip(p.astype(jnp.int32), 0, T - 2)
                    t = p - i0.astype(jnp.float32)
                    g0 = plsc.load_gather(vs, [i0])
                    g1 = plsc.load_gather(vs, [i0 + 1])
                    pn = p + DT * (g0 * (1.0 - t) + g1 * t)
                    phi[pl.ds(off, L)] = jnp.clip(pn, 0.0, float(T - 1))
                return c2
            lax.fori_loop(0, T // (4 * L), grp, 0)
            return carry
        lax.fori_loop(0, NIT, euler_body, 0)
        pltpu.sync_copy(phi, shared.at[lr])
    plsc.subcore_barrier()

    b_loc = s // 2
    h = s % 2
    b = c * (B // NC) + b_loc
    for k in range(K):
        pltpu.sync_copy(shared.at[4 * b_loc + k], f4.at[k])

    def comp_body(g, carry):
        pos = f4[0, pl.ds(h * NPAIR + g * L, L)]
        for k in range(1, K):
            i0 = jnp.clip(pos.astype(jnp.int32), 0, T - 2)
            t = pos - i0.astype(jnp.float32)
            kvec = jnp.full((L,), k, jnp.int32)
            vm1 = plsc.load_gather(f4, [kvec, jnp.maximum(i0 - 1, 0)])
            v0 = plsc.load_gather(f4, [kvec, i0])
            v1 = plsc.load_gather(f4, [kvec, i0 + 1])
            v2 = plsc.load_gather(f4, [kvec, jnp.minimum(i0 + 2, T - 1)])
            t2 = t * t
            t3 = t2 * t
            w0 = -0.5 * t3 + t2 - 0.5 * t
            w1 = 1.5 * t3 - 2.5 * t2 + 1.0
            w2 = -1.5 * t3 + 2.0 * t2 + 0.5 * t
            w3 = 0.5 * t3 - 0.5 * t2
            pos = w0 * vm1 + w1 * v0 + w2 * v1 + w3 * v2
        func_v[pl.ds(g * L, L)] = pos
        i0 = jnp.clip(pos.astype(jnp.int32), 0, T - 2)
        t = pos - i0.astype(jnp.float32)
        t2 = t * t
        t3 = t2 * t
        wbuf[0, pl.ds(g * L, L)] = -0.5 * t3 + t2 - 0.5 * t
        wbuf[1, pl.ds(g * L, L)] = 1.5 * t3 - 2.5 * t2 + 1.0
        wbuf[2, pl.ds(g * L, L)] = -1.5 * t3 + 2.0 * t2 + 0.5 * t
        wbuf[3, pl.ds(g * L, L)] = 0.5 * t3 - 0.5 * t2
        base_row = b * T
        m = g // 2
        qoff = (g % 2) * L
        idx_all[m, pl.ds(qoff, L)] = base_row + jnp.maximum(i0 - 1, 0)
        idx_all[m, pl.ds(PCHUNK + qoff, L)] = base_row + i0
        idx_all[m, pl.ds(2 * PCHUNK + qoff, L)] = base_row + i0 + 1
        idx_all[m, pl.ds(3 * PCHUNK + qoff, L)] = (
            base_row + jnp.minimum(i0 + 2, T - 1))
        return carry
    lax.fori_loop(0, NPAIR // L, comp_body, 0)
    pltpu.sync_copy(func_v, func_hbm.at[b, pl.ds(h * NPAIR, NPAIR)])

    out_base = b * T + h * NPAIR

    def chunk_body(m, carry):
        pltpu.async_copy(xs_hbm.at[idx_all.at[m]], grows, sem).wait()

        def pair_body(q, c2):
            qi = m * PCHUNK + q
            qsplat = jnp.full((L,), 0, jnp.int32) + qi
            w0v = plsc.load_gather(wbuf, [jnp.full((L,), 0, jnp.int32), qsplat])
            w1v = plsc.load_gather(wbuf, [jnp.full((L,), 1, jnp.int32), qsplat])
            w2v = plsc.load_gather(wbuf, [jnp.full((L,), 2, jnp.int32), qsplat])
            w3v = plsc.load_gather(wbuf, [jnp.full((L,), 3, jnp.int32), qsplat])
            for cc in range(C // L):
                sl = pl.ds(cc * L, L)
                o = (w0v * grows[q, sl] + w1v * grows[PCHUNK + q, sl]
                     + w2v * grows[2 * PCHUNK + q, sl]
                     + w3v * grows[3 * PCHUNK + q, sl])
                orow[q, sl] = o
            return c2
        lax.fori_loop(0, PCHUNK, pair_body, 0)
        pltpu.sync_copy(orow, ys_hbm.at[pl.ds(out_base + m * PCHUNK, PCHUNK)])
        return carry
    lax.fori_loop(0, NCHUNK, chunk_body, 0)


def kernel(xs, ds, is_inference):
    xs_flat = xs.reshape(B * T, C)
    v64 = jnp.swapaxes(ds, -1, -2).reshape(NROWS, T)
    mesh = plsc.VectorSubcoreMesh(core_axis_name="c", subcore_axis_name="s")
    f = pl.kernel(
        _sc_body,
        out_type=(jax.ShapeDtypeStruct((B * T, C), jnp.float32),
                  jax.ShapeDtypeStruct((B, T), jnp.float32)),
        mesh=mesh,
        scratch_types=[
            pltpu.VMEM((T,), jnp.float32),
            pltpu.VMEM((T,), jnp.float32),
            pltpu.VMEM((T,), jnp.float32),
            pltpu.VMEM((K, T), jnp.float32),
            pltpu.VMEM((NPAIR,), jnp.float32),
            pltpu.VMEM((K, NPAIR), jnp.float32),
            pltpu.VMEM((NCHUNK, 4 * PCHUNK), jnp.int32),
            pltpu.VMEM((4 * PCHUNK, C), jnp.float32),
            pltpu.VMEM((PCHUNK, C), jnp.float32),
            pltpu.VMEM_SHARED((2 * NS, T), jnp.float32),
            pltpu.SemaphoreType.DMA,
        ],
        compiler_params=pltpu.CompilerParams(needs_layout_passes=False),
    )
    ys_flat, func = f(v64, xs_flat)
    return ys_flat.reshape(B, T, C), func
